# baseline (device time: 60256 ns/iter reference)
import jax
import jax.numpy as jnp
from jax import lax
from jax.experimental import pallas as pl
from jax.experimental.pallas import tpu as pltpu

N_DEV = 32
B, SQ, DM = 2, 256, 512
H, DH = 4, 64
HALO = 128
G = 32
W = HALO + SQ + HALO
BIG = 10 ** 9
SCALE = 0.125
NEG = -1e9
BF = jnp.bfloat16
F32 = jnp.float32


def kernel(x, Wq, K_ext, V_ext, Wo):
    def body(x_ref, wq_ref, k_ref, v_ref, wo_ref, out_ref,
             ws, bc, part, part_rx, ctxb,
             hs, hr, bs, bcr, ps, prx):
        p = lax.axis_index("i")

        def halo_right():
            return pltpu.make_async_remote_copy(
                src_ref=ws.at[:, :, SQ:HALO + SQ],
                dst_ref=ws.at[:, :, 0:HALO],
                send_sem=hs.at[1], recv_sem=hr.at[0],
                device_id=((p + 1) % N_DEV,),
                device_id_type=pl.DeviceIdType.MESH)

        def halo_left():
            return pltpu.make_async_remote_copy(
                src_ref=ws.at[:, :, HALO:2 * HALO],
                dst_ref=ws.at[:, :, HALO + SQ:HALO + SQ + HALO],
                send_sem=hs.at[0], recv_sem=hr.at[1],
                device_id=((p - 1) % N_DEV,),
                device_id_type=pl.DeviceIdType.MESH)

        is_leader = (p % 8 == 0) & (p > 0)
        my_leader = (p // 8) * 8

        def bc_send(i, t):
            return pltpu.make_async_remote_copy(
                src_ref=bc, dst_ref=bc,
                send_sem=bs.at[i], recv_sem=bcr.at[0],
                device_id=(t,), device_id_type=pl.DeviceIdType.MESH)

        def part_send(slot, t):
            return pltpu.make_async_remote_copy(
                src_ref=part, dst_ref=part_rx.at[pl.ds(slot, 1)],
                send_sem=ps.at[0], recv_sem=prx.at[0],
                device_id=(t,), device_id_type=pl.DeviceIdType.MESH)


        ws[0, :, HALO:HALO + SQ] = k_ref[...].astype(BF)
        ws[1, :, HALO:HALO + SQ] = v_ref[...].astype(BF)

        @pl.when(p == 0)
        def _():
            z = jnp.zeros((B, HALO, H, DH), BF)
            ws[0, :, 0:HALO] = z
            ws[1, :, 0:HALO] = z

        @pl.when(p == N_DEV - 1)
        def _():
            z = jnp.zeros((B, HALO, H, DH), BF)
            ws[0, :, HALO + SQ:HALO + SQ + HALO] = z
            ws[1, :, HALO + SQ:HALO + SQ + HALO] = z

        @pl.when(p < N_DEV - 1)
        def _():
            halo_right().start()

        @pl.when(p > 0)
        def _():
            halo_left().start()

        wq = wq_ref[...].astype(BF)
        qs = []
        for b in range(B):
            qb = lax.dot_general(
                x_ref[b].astype(BF), wq, (((1,), (0,)), ((), ())),
                preferred_element_type=F32)
            qs.append(qb.reshape(SQ, H, DH))

        @pl.when(p == 0)
        def _():
            bc[0] = k_ref[:, 0:G].astype(BF)
            bc[1] = v_ref[:, 0:G].astype(BF)
            for b in range(B):
                bc[2, b] = qs[b][0:G].astype(BF)
            for i, t in enumerate([8, 16, 24]):
                bc_send(i, t).start()

        @pl.when(p != 0)
        def _():
            bc_send(0, 0).wait_recv()

        @pl.when(is_leader)
        def _():
            for i in range(1, 8):
                bc_send(i - 1, p + i).start()

        @pl.when(p == 8)
        def _():
            for j, t in enumerate(range(1, 8)):
                bc_send(7 + j, t).start()

        for b in range(B):
            for h in range(H):
                q32 = bc[2, b, :, h, :]
                kb = k_ref[b, :, h, :].astype(BF)
                s = lax.dot_general(
                    q32, kb, (((1,), (1,)), ((), ())),
                    preferred_element_type=F32) * SCALE
                m = jnp.max(s, axis=1, keepdims=True)
                e = jnp.exp(s - m)
                l = jnp.sum(e, axis=1, keepdims=True)
                vb = v_ref[b, :, h, :].astype(BF)
                pc = lax.dot_general(
                    e.astype(BF), vb, (((1,), (0,)), ((), ())),
                    preferred_element_type=F32)
                part[0, b, h, :, 0:DH] = pc.astype(BF)
                part[0, b, h, :, DH:DH + 1] = m.astype(BF)
                part[0, b, h, :, DH + 1:DH + 2] = l.astype(BF)

        @pl.when((p != 0) & ~is_leader)
        def _():
            part_send(p % 8, my_leader).start()

        @pl.when((p == 0) | is_leader)
        def _():
            part_rx[0] = part[0]

        @pl.when(is_leader)
        def _():
            rd = part_send(0, 0)
            for _ in range(7):
                rd.wait_recv()
            pr = part_rx[0:8].astype(F32).reshape(8, B * H * G, DH + 2)
            M = jnp.max(pr[:, :, DH:DH + 1], axis=0, keepdims=True)
            a = jnp.exp(pr[:, :, DH:DH + 1] - M)
            Lc = jnp.sum(a * pr[:, :, DH + 1:DH + 2], axis=0)
            Cc = jnp.sum(a * pr[:, :, 0:DH], axis=0)
            comb = jnp.concatenate([Cc, M[0], Lc], axis=1)
            part[...] = comb.reshape(1, B, H, G, DH + 2).astype(BF)
            part_send(7 + p // 8, 0).start()

        @pl.when(p > 0)
        def _():
            halo_right().wait_recv()

        @pl.when(p < N_DEV - 1)
        def _():
            halo_left().wait_recv()

        qi = lax.broadcasted_iota(jnp.int32, (SQ, W), 0) + SQ * p
        kl = lax.broadcasted_iota(jnp.int32, (1, HALO), 1) + SQ * p - HALO
        kl = jnp.where(p > 0, kl, BIG)
        kc = lax.broadcasted_iota(jnp.int32, (1, SQ), 1) + SQ * p
        kr = lax.broadcasted_iota(jnp.int32, (1, HALO), 1) + SQ * (p + 1)
        kr = jnp.where(p < N_DEV - 1, kr, BIG)
        kcol = jnp.concatenate([kl, kc, kr], axis=1)
        mask = ((jnp.abs(qi - kcol) <= HALO) | (kcol < G) | (qi < G))

        for b in range(B):
            for h in range(H):
                qbh = qs[b][:, h, :].astype(BF)
                kws = ws[0, b, :, h, :]
                s = lax.dot_general(
                    qbh, kws, (((1,), (1,)), ((), ())),
                    preferred_element_type=F32) * SCALE
                s = jnp.where(mask, s, NEG)
                m1 = jnp.max(s, axis=1, keepdims=True)
                e = jnp.exp(s - m1)
                l1 = jnp.sum(e, axis=1, keepdims=True)
                vws = ws[1, b, :, h, :]
                c1 = lax.dot_general(
                    e.astype(BF), vws, (((1,), (0,)), ((), ())),
                    preferred_element_type=F32)

                kg32 = bc[0, b, :, h, :]
                vg32 = bc[1, b, :, h, :]
                s2 = lax.dot_general(
                    qbh, kg32, (((1,), (1,)), ((), ())),
                    preferred_element_type=F32) * SCALE
                s2 = jnp.where(p > 0, s2, NEG)
                m2 = jnp.max(s2, axis=1, keepdims=True)
                e2 = jnp.exp(s2 - m2)
                l2 = jnp.sum(e2, axis=1, keepdims=True)
                c2 = lax.dot_general(
                    e2.astype(BF), vg32, (((1,), (0,)), ((), ())),
                    preferred_element_type=F32)

                mm = jnp.maximum(m1, m2)
                a1 = jnp.exp(m1 - mm)
                a2 = jnp.exp(m2 - mm)
                den = l1 * a1 + l2 * a2
                ctxb[b, :, h, :] = (c1 * a1 + c2 * a2) / den

        wo = wo_ref[...].astype(BF)
        for b in range(B):
            cb = ctxb[b].reshape(SQ, H * DH).astype(BF)
            out_ref[b] = lax.dot_general(
                cb, wo, (((1,), (0,)), ((), ())),
                preferred_element_type=F32)

        @pl.when(p == 0)
        def _():
            rd = part_send(0, 0)
            for _ in range(10):
                rd.wait_recv()
            pr = part_rx[0:11].astype(F32).reshape(11, B * H * G, DH + 2)
            M = jnp.max(pr[:, :, DH:DH + 1], axis=0, keepdims=True)
            a = jnp.exp(pr[:, :, DH:DH + 1] - M)
            L = jnp.sum(a * pr[:, :, DH + 1:DH + 2], axis=0)
            C = jnp.sum(a * pr[:, :, 0:DH], axis=0) / L
            C = C.reshape(B, H, G, DH)
            for h in range(H):
                ctxb[:, 0:G, h, :] = C[:, h].astype(F32)
            for b in range(B):
                cg = ctxb[b, 0:G].reshape(G, H * DH).astype(BF)
                out_ref[b, 0:G] = lax.dot_general(
                    cg, wo, (((1,), (0,)), ((), ())),
                    preferred_element_type=F32)

        @pl.when(p > 0)
        def _():
            halo_left().wait_send()

        @pl.when(p < N_DEV - 1)
        def _():
            halo_right().wait_send()

        @pl.when(p == 0)
        def _():
            for i in range(3):
                bc_send(i, 0).wait_send()

        @pl.when(is_leader)
        def _():
            for i in range(7):
                bc_send(i, 0).wait_send()

        @pl.when(p == 8)
        def _():
            for i in range(7, 14):
                bc_send(i, 0).wait_send()

        @pl.when(p != 0)
        def _():
            part_send(0, 0).wait_send()

    return pl.pallas_call(
        body,
        out_shape=jax.ShapeDtypeStruct((B, SQ, DM), F32),
        in_specs=[pl.BlockSpec(memory_space=pltpu.VMEM)] * 5,
        out_specs=pl.BlockSpec(memory_space=pltpu.VMEM),
        scratch_shapes=[
            pltpu.VMEM((2, B, W, H, DH), BF),
            pltpu.VMEM((3, B, G, H, DH), BF),
            pltpu.VMEM((1, B, H, G, DH + 2), BF),
            pltpu.VMEM((N_DEV, B, H, G, DH + 2), BF),
            pltpu.VMEM((B, SQ, H, DH), F32),
            pltpu.SemaphoreType.DMA((2,)),
            pltpu.SemaphoreType.DMA((2,)),
            pltpu.SemaphoreType.DMA((14,)),
            pltpu.SemaphoreType.DMA((1,)),
            pltpu.SemaphoreType.DMA((1,)),
            pltpu.SemaphoreType.DMA((1,)),
        ],
    )(x, Wq, K_ext, V_ext, Wo)


# device time: 56255 ns/iter; 1.0711x vs baseline; 1.0711x over previous
import jax
import jax.numpy as jnp
from jax import lax
from jax.experimental import pallas as pl
from jax.experimental.pallas import tpu as pltpu

N_DEV = 32
B, SQ, DM = 2, 256, 512
H, DH = 4, 64
HALO = 128
G = 32
W = HALO + SQ + HALO
BIG = 10 ** 9
SCALE = 0.125
NEG = -1e9
BF = jnp.bfloat16
F32 = jnp.float32


def kernel(x, Wq, K_ext, V_ext, Wo):
    def body(x_ref, wq_ref, k_ref, v_ref, wo_ref, out_ref,
             ws, bc, part, part_rx, ctxb,
             hs, hr, bs, bcr, ps, prx):
        p = lax.axis_index("i")

        def halo_right():
            return pltpu.make_async_remote_copy(
                src_ref=ws.at[:, :, SQ:HALO + SQ],
                dst_ref=ws.at[:, :, 0:HALO],
                send_sem=hs.at[1], recv_sem=hr.at[0],
                device_id=((p + 1) % N_DEV,),
                device_id_type=pl.DeviceIdType.MESH)

        def halo_left():
            return pltpu.make_async_remote_copy(
                src_ref=ws.at[:, :, HALO:2 * HALO],
                dst_ref=ws.at[:, :, HALO + SQ:HALO + SQ + HALO],
                send_sem=hs.at[0], recv_sem=hr.at[1],
                device_id=((p - 1) % N_DEV,),
                device_id_type=pl.DeviceIdType.MESH)

        is_leader = (p % 8 == 0) & (p > 0)
        my_leader = (p // 8) * 8

        def bc_send(i, t):
            return pltpu.make_async_remote_copy(
                src_ref=bc, dst_ref=bc,
                send_sem=bs.at[i], recv_sem=bcr.at[0],
                device_id=(t,), device_id_type=pl.DeviceIdType.MESH)

        def part_send(slot, t):
            return pltpu.make_async_remote_copy(
                src_ref=part, dst_ref=part_rx.at[pl.ds(slot, 1)],
                send_sem=ps.at[0], recv_sem=prx.at[0],
                device_id=(t,), device_id_type=pl.DeviceIdType.MESH)


        ws[0, :, HALO:HALO + SQ] = k_ref[...].astype(BF)
        ws[1, :, HALO:HALO + SQ] = v_ref[...].astype(BF)

        @pl.when(p == 0)
        def _():
            z = jnp.zeros((B, HALO, H, DH), BF)
            ws[0, :, 0:HALO] = z
            ws[1, :, 0:HALO] = z

        @pl.when(p == N_DEV - 1)
        def _():
            z = jnp.zeros((B, HALO, H, DH), BF)
            ws[0, :, HALO + SQ:HALO + SQ + HALO] = z
            ws[1, :, HALO + SQ:HALO + SQ + HALO] = z

        @pl.when(p < N_DEV - 1)
        def _():
            halo_right().start()

        @pl.when(p > 0)
        def _():
            halo_left().start()

        wq = wq_ref[...].astype(BF)
        qs = []
        for b in range(B):
            qb = lax.dot_general(
                x_ref[b].astype(BF), wq, (((1,), (0,)), ((), ())),
                preferred_element_type=F32)
            qs.append(qb.reshape(SQ, H, DH))

        @pl.when(p == 0)
        def _():
            bc[0] = k_ref[:, 0:G].astype(BF)
            bc[1] = v_ref[:, 0:G].astype(BF)
            for b in range(B):
                bc[2, b] = qs[b][0:G].astype(BF)
            for i, t in enumerate([8, 16, 24, 1, 2, 3, 4, 5, 6, 7]):
                bc_send(i, t).start()

        @pl.when(p != 0)
        def _():
            bc_send(0, 0).wait_recv()

        @pl.when(is_leader)
        def _():
            for i in range(1, 8):
                bc_send(i - 1, p + i).start()

        for b in range(B):
            for h in range(H):
                q32 = bc[2, b, :, h, :]
                kb = k_ref[b, :, h, :].astype(BF)
                s = lax.dot_general(
                    q32, kb, (((1,), (1,)), ((), ())),
                    preferred_element_type=F32) * SCALE
                m = jnp.max(s, axis=1, keepdims=True)
                e = jnp.exp(s - m)
                l = jnp.sum(e, axis=1, keepdims=True)
                vb = v_ref[b, :, h, :].astype(BF)
                pc = lax.dot_general(
                    e.astype(BF), vb, (((1,), (0,)), ((), ())),
                    preferred_element_type=F32)
                part[0, b, h, :, 0:DH] = pc.astype(BF)
                part[0, b, h, :, DH:DH + 1] = m.astype(BF)
                part[0, b, h, :, DH + 1:DH + 2] = l.astype(BF)

        @pl.when((p != 0) & ~is_leader)
        def _():
            part_send(p % 8, my_leader).start()

        @pl.when((p == 0) | is_leader)
        def _():
            part_rx[0] = part[0]

        @pl.when(is_leader)
        def _():
            rd = part_send(0, 0)
            for _ in range(7):
                rd.wait_recv()
            pr = part_rx[0:8].astype(F32).reshape(8, B * H * G, DH + 2)
            M = jnp.max(pr[:, :, DH:DH + 1], axis=0, keepdims=True)
            a = jnp.exp(pr[:, :, DH:DH + 1] - M)
            Lc = jnp.sum(a * pr[:, :, DH + 1:DH + 2], axis=0)
            Cc = jnp.sum(a * pr[:, :, 0:DH], axis=0)
            comb = jnp.concatenate([Cc, M[0], Lc], axis=1)
            part[...] = comb.reshape(1, B, H, G, DH + 2).astype(BF)
            part_send(7 + p // 8, 0).start()

        @pl.when(p > 0)
        def _():
            halo_right().wait_recv()

        @pl.when(p < N_DEV - 1)
        def _():
            halo_left().wait_recv()

        qi = lax.broadcasted_iota(jnp.int32, (SQ, W), 0) + SQ * p
        kl = lax.broadcasted_iota(jnp.int32, (1, HALO), 1) + SQ * p - HALO
        kl = jnp.where(p > 0, kl, BIG)
        kc = lax.broadcasted_iota(jnp.int32, (1, SQ), 1) + SQ * p
        kr = lax.broadcasted_iota(jnp.int32, (1, HALO), 1) + SQ * (p + 1)
        kr = jnp.where(p < N_DEV - 1, kr, BIG)
        kcol = jnp.concatenate([kl, kc, kr], axis=1)
        mask = ((jnp.abs(qi - kcol) <= HALO) | (kcol < G) | (qi < G))

        for b in range(B):
            for h in range(H):
                qbh = qs[b][:, h, :].astype(BF)
                kws = ws[0, b, :, h, :]
                s = lax.dot_general(
                    qbh, kws, (((1,), (1,)), ((), ())),
                    preferred_element_type=F32) * SCALE
                s = jnp.where(mask, s, NEG)
                m1 = jnp.max(s, axis=1, keepdims=True)
                e = jnp.exp(s - m1)
                l1 = jnp.sum(e, axis=1, keepdims=True)
                vws = ws[1, b, :, h, :]
                c1 = lax.dot_general(
                    e.astype(BF), vws, (((1,), (0,)), ((), ())),
                    preferred_element_type=F32)

                kg32 = bc[0, b, :, h, :]
                vg32 = bc[1, b, :, h, :]
                s2 = lax.dot_general(
                    qbh, kg32, (((1,), (1,)), ((), ())),
                    preferred_element_type=F32) * SCALE
                s2 = jnp.where(p > 0, s2, NEG)
                m2 = jnp.max(s2, axis=1, keepdims=True)
                e2 = jnp.exp(s2 - m2)
                l2 = jnp.sum(e2, axis=1, keepdims=True)
                c2 = lax.dot_general(
                    e2.astype(BF), vg32, (((1,), (0,)), ((), ())),
                    preferred_element_type=F32)

                mm = jnp.maximum(m1, m2)
                a1 = jnp.exp(m1 - mm)
                a2 = jnp.exp(m2 - mm)
                den = l1 * a1 + l2 * a2
                ctxb[b, :, h, :] = (c1 * a1 + c2 * a2) / den

        wo = wo_ref[...].astype(BF)
        for b in range(B):
            cb = ctxb[b].reshape(SQ, H * DH).astype(BF)
            out_ref[b] = lax.dot_general(
                cb, wo, (((1,), (0,)), ((), ())),
                preferred_element_type=F32)

        @pl.when(p == 0)
        def _():
            rd = part_send(0, 0)
            for _ in range(10):
                rd.wait_recv()
            pr = part_rx[0:11].astype(F32).reshape(11, B * H * G, DH + 2)
            M = jnp.max(pr[:, :, DH:DH + 1], axis=0, keepdims=True)
            a = jnp.exp(pr[:, :, DH:DH + 1] - M)
            L = jnp.sum(a * pr[:, :, DH + 1:DH + 2], axis=0)
            C = jnp.sum(a * pr[:, :, 0:DH], axis=0) / L
            C = C.reshape(B, H, G, DH)
            for h in range(H):
                ctxb[:, 0:G, h, :] = C[:, h].astype(F32)
            for b in range(B):
                cg = ctxb[b, 0:G].reshape(G, H * DH).astype(BF)
                out_ref[b, 0:G] = lax.dot_general(
                    cg, wo, (((1,), (0,)), ((), ())),
                    preferred_element_type=F32)

        @pl.when(p > 0)
        def _():
            halo_left().wait_send()

        @pl.when(p < N_DEV - 1)
        def _():
            halo_right().wait_send()

        @pl.when(p == 0)
        def _():
            for i in range(10):
                bc_send(i, 0).wait_send()

        @pl.when(is_leader)
        def _():
            for i in range(7):
                bc_send(i, 0).wait_send()

        @pl.when(p != 0)
        def _():
            part_send(0, 0).wait_send()

    return pl.pallas_call(
        body,
        out_shape=jax.ShapeDtypeStruct((B, SQ, DM), F32),
        in_specs=[pl.BlockSpec(memory_space=pltpu.VMEM)] * 5,
        out_specs=pl.BlockSpec(memory_space=pltpu.VMEM),
        scratch_shapes=[
            pltpu.VMEM((2, B, W, H, DH), BF),
            pltpu.VMEM((3, B, G, H, DH), BF),
            pltpu.VMEM((1, B, H, G, DH + 2), BF),
            pltpu.VMEM((N_DEV, B, H, G, DH + 2), BF),
            pltpu.VMEM((B, SQ, H, DH), F32),
            pltpu.SemaphoreType.DMA((2,)),
            pltpu.SemaphoreType.DMA((2,)),
            pltpu.SemaphoreType.DMA((14,)),
            pltpu.SemaphoreType.DMA((1,)),
            pltpu.SemaphoreType.DMA((1,)),
            pltpu.SemaphoreType.DMA((1,)),
        ],
    )(x, Wq, K_ext, V_ext, Wo)
